# baseline (speedup 1.0000x reference)
import functools

import jax
import jax.numpy as jnp
from jax import lax
from jax.experimental import pallas as pl
from jax.experimental.pallas import tpu as pltpu
from jax.experimental.pallas import tpu_sc as plsc

N = 10000
E = 320000
IN = 128
H = 8
F = 16
HF = H * F
NEG = 0.2

NTILES = 32
K = 128
EDGES_PER_TILE = 10240
EP = NTILES * EDGES_PER_TILE
NCH = EDGES_PER_TILE // K
NP = 10112
ROWS_PER_TILE = NP // 16


def _leaky(v):
    return jnp.where(v > 0, v, NEG * v)


def _pre_body(x_ref, w_ref, ms_ref, md_ref, h_ref, stab_ref, dtab_ref):
    h = jnp.dot(x_ref[...], w_ref[...], preferred_element_type=jnp.float32)
    h_ref[...] = h
    als = jnp.dot(h, ms_ref[...], preferred_element_type=jnp.float32)
    ald = jnp.dot(h, md_ref[...], preferred_element_type=jnp.float32)
    es = _leaky(als + ald)
    z8 = jnp.zeros_like(als)
    stab_ref[...] = jnp.concatenate([als, z8], axis=1)
    dtab_ref[...] = jnp.concatenate([ald, z8, es, z8], axis=1)


def _tc_prologue(x, W, Ms, Md):
    return pl.pallas_call(
        _pre_body,
        out_shape=(
            jax.ShapeDtypeStruct((N, HF), jnp.float32),
            jax.ShapeDtypeStruct((N, 16), jnp.float32),
            jax.ShapeDtypeStruct((N, 32), jnp.float32),
        ),
    )(x, W, Ms, Md)


def _bcast_lane(v, j):
    idx = jnp.broadcast_to(jnp.int32(j), (16,))
    return lax.gather(
        v, idx[:, None],
        dimension_numbers=lax.GatherDimensionNumbers(
            offset_dims=(), collapsed_slice_dims=(0,), start_index_map=(0,)),
        slice_sizes=(1,),
        mode=lax.GatherScatterMode.PROMISE_IN_BOUNDS)


def _sc_edge_kernel(h_hbm, stab_hbm, dtab_hbm, srcs_hbm, dsts_hbm,
                    outp_hbm, denp_hbm,
                    idxs_v, idxd_v, hbuf, sbuf, dbuf, exbuf,
                    out_acc, den_acc):
    c = lax.axis_index("c")
    s = lax.axis_index("s")

    def _zrow(i, _):
        for j in range(HF // 16):
            hbuf[i, pl.ds(16 * j, 16)] = jnp.zeros((16,), jnp.float32)
        exbuf[i, :] = jnp.zeros((16,), jnp.float32)
        return _
    lax.fori_loop(0, K, _zrow, None)

    r0 = s * ROWS_PER_TILE
    done = 0
    for rows in (K, K, K, K, ROWS_PER_TILE - 4 * K):
        pltpu.sync_copy(hbuf.at[pl.ds(0, rows)],
                        out_acc.at[pl.ds(r0 + done, rows)])
        pltpu.sync_copy(exbuf.at[pl.ds(0, rows)],
                        den_acc.at[pl.ds(r0 + done, rows)])
        done += rows
    plsc.subcore_barrier()

    tile_base = (c * 16 + s) * EDGES_PER_TILE

    def _chunk(g, _):
        base = tile_base + g * K
        pltpu.sync_copy(srcs_hbm.at[pl.ds(base, K)], idxs_v)
        pltpu.sync_copy(dsts_hbm.at[pl.ds(base, K)], idxd_v)
        pltpu.sync_copy(h_hbm.at[idxs_v], hbuf)
        pltpu.sync_copy(stab_hbm.at[idxs_v], sbuf)
        pltpu.sync_copy(dtab_hbm.at[idxd_v], dbuf)

        def _edge(i, _):
            srow = sbuf[i, :]
            ad16 = dbuf[i, pl.ds(0, 16)]
            es16 = dbuf[i, pl.ds(16, 16)]
            t = srow + ad16
            e = jnp.where(t > 0, t, NEG * t)
            ex = jnp.exp(e - es16)
            exbuf[i, :] = ex
            for j in range(H):
                b = _bcast_lane(ex, j)
                sl = pl.ds(16 * j, 16)
                hbuf[i, sl] = hbuf[i, sl] * b
            return _
        lax.fori_loop(0, K, _edge, None)

        pltpu.sync_copy(exbuf, den_acc.at[idxd_v], add=True)
        pltpu.sync_copy(hbuf, out_acc.at[idxd_v], add=True)
        return _
    lax.fori_loop(0, NCH, _chunk, None)

    plsc.subcore_barrier()
    pltpu.sync_copy(out_acc.at[pl.ds(r0, ROWS_PER_TILE)],
                    outp_hbm.at[c, pl.ds(r0, ROWS_PER_TILE)])
    pltpu.sync_copy(den_acc.at[pl.ds(r0, ROWS_PER_TILE)],
                    denp_hbm.at[c, pl.ds(r0, ROWS_PER_TILE)])


def _sc_edge_pass(h, stab, dtab, srcs, dsts):
    mesh = plsc.VectorSubcoreMesh(core_axis_name="c", subcore_axis_name="s")
    run = functools.partial(
        pl.kernel,
        mesh=mesh,
        compiler_params=pltpu.CompilerParams(use_tc_tiling_on_sc=False),
        out_type=(
            jax.ShapeDtypeStruct((2, NP, HF), jnp.float32),
            jax.ShapeDtypeStruct((2, NP, 16), jnp.float32),
        ),
        scratch_types=[
            pltpu.VMEM((K,), jnp.int32),
            pltpu.VMEM((K,), jnp.int32),
            pltpu.VMEM((K, HF), jnp.float32),
            pltpu.VMEM((K, 16), jnp.float32),
            pltpu.VMEM((K, 32), jnp.float32),
            pltpu.VMEM((K, 16), jnp.float32),
            pltpu.VMEM_SHARED((NP, HF), jnp.float32),
            pltpu.VMEM_SHARED((NP, 16), jnp.float32),
        ],
    )(_sc_edge_kernel)
    return run(h, stab, dtab, srcs, dsts)


def _post_body(outp_ref, denp_ref, h_ref, bias_ref, gamma_ref, beta_ref,
               b128_ref, o_ref):
    acc = outp_ref[0, :N, :] + outp_ref[1, :N, :] + h_ref[...]
    den = denp_ref[0, :N, :] + denp_ref[1, :N, :] + (1.0 + 1e-16)
    dinv = 1.0 / den
    dinv128 = jnp.dot(dinv, b128_ref[...],
                      preferred_element_type=jnp.float32)
    y = acc * dinv128 + bias_ref[...]
    mean = jnp.mean(y, axis=0, keepdims=True)
    var = jnp.mean((y - mean) ** 2, axis=0, keepdims=True)
    yn = (y - mean) / jnp.sqrt(var + 1e-5) * gamma_ref[...] + beta_ref[...]
    o_ref[...] = jnp.where(yn > 0, yn, NEG * yn)


def _tc_epilogue(outp, denp, h, bias, gamma, beta, B128):
    return pl.pallas_call(
        _post_body,
        out_shape=jax.ShapeDtypeStruct((N, HF), jnp.float32),
    )(outp, denp, h, bias, gamma, beta, B128)


def kernel(x, edge_index, W, a_src, a_dst, bias, gamma, beta):
    r = jnp.arange(HF, dtype=jnp.int32)
    Ms = jnp.zeros((HF, H), jnp.float32).at[r, r // F].set(a_src.reshape(-1))
    Md = jnp.zeros((HF, H), jnp.float32).at[r, r // F].set(a_dst.reshape(-1))
    B128 = jnp.zeros((16, HF), jnp.float32).at[r // F, r].set(1.0)

    h, stab, dtab = _tc_prologue(x, W, Ms, Md)

    pad = EP - E
    srcs = jnp.concatenate([edge_index[0], jnp.zeros((pad,), jnp.int32)])
    dsts = jnp.concatenate([edge_index[1], jnp.full((pad,), N, jnp.int32)])

    outp, denp = _sc_edge_pass(h, stab, dtab, srcs, dsts)

    return _tc_epilogue(outp, denp, h, bias.reshape(1, HF),
                        gamma.reshape(1, HF), beta.reshape(1, HF), B128)


# speedup vs baseline: 2.7745x; 2.7745x over previous
import functools

import jax
import jax.numpy as jnp
from jax import lax
from jax.experimental import pallas as pl
from jax.experimental.pallas import tpu as pltpu
from jax.experimental.pallas import tpu_sc as plsc

N = 10000
E = 320000
IN = 128
H = 8
F = 16
HF = H * F
TW = HF + 16
NEG = 0.2

NTILES = 32
K = 112
NCH = 90
EDGES_PER_TILE = NCH * K
EP = NTILES * EDGES_PER_TILE
NP = 10112
ROWS_PER_TILE = NP // 16


def _leaky(v):
    return jnp.where(v > 0, v, NEG * v)


def _pre_body(x_ref, w_ref, ms_ref, md_ref, stab_ref, dtab_ref):
    h = jnp.dot(x_ref[...], w_ref[...], preferred_element_type=jnp.float32)
    als = jnp.dot(h, ms_ref[...], preferred_element_type=jnp.float32)
    ald = jnp.dot(h, md_ref[...], preferred_element_type=jnp.float32)
    es = _leaky(als + ald)
    stab_ref[...] = jnp.concatenate([h, als, als], axis=1)
    dtab_ref[...] = jnp.concatenate([ald, es], axis=1)


def _tc_prologue(x, W, Ms, Md):
    return pl.pallas_call(
        _pre_body,
        out_shape=(
            jax.ShapeDtypeStruct((N, TW), jnp.float32),
            jax.ShapeDtypeStruct((N, 16), jnp.float32),
        ),
    )(x, W, Ms, Md)


def _lane_gather(v, idx):
    return lax.gather(
        v, idx[:, None],
        dimension_numbers=lax.GatherDimensionNumbers(
            offset_dims=(), collapsed_slice_dims=(0,), start_index_map=(0,)),
        slice_sizes=(1,),
        mode=lax.GatherScatterMode.PROMISE_IN_BOUNDS)


def _bcast_lane(v, j):
    return _lane_gather(v, jnp.broadcast_to(jnp.int32(j), (16,)))


def _sc_edge_kernel(stab_hbm, dtab_hbm, srcs_hbm, dsts_hbm,
                    outp_hbm,
                    sidx0, sidx1, didx0, didx1, sg0, sg1, dg0, dg1,
                    gsem0, gsem1, isem0, isem1,
                    out_acc):
    c = lax.axis_index("c")
    s = lax.axis_index("s")
    t = c * 16 + s
    tile_base = t * EDGES_PER_TILE

    def _zrow(i, _):
        for j in range(TW // 16):
            sg0[i, pl.ds(16 * j, 16)] = jnp.zeros((16,), jnp.float32)
        return _
    lax.fori_loop(0, K, _zrow, None)

    r0 = s * ROWS_PER_TILE
    done = 0
    for rows in (K, K, K, K, K, ROWS_PER_TILE - 5 * K):
        pltpu.sync_copy(sg0.at[pl.ds(0, rows)],
                        out_acc.at[pl.ds(r0 + done, rows)])
        done += rows
    plsc.subcore_barrier()

    sgs = (sg0, sg1)
    dgs = (dg0, dg1)
    sidxs = (sidx0, sidx1)
    didxs = (didx0, didx1)
    gsems = (gsem0, gsem1)
    isems = (isem0, isem1)

    def _issue_idx(g, b, copy=pltpu.async_copy):
        base = tile_base + g * K
        copy(srcs_hbm.at[pl.ds(base, K)], sidxs[b], isems[b])
        copy(dsts_hbm.at[pl.ds(base, K)], didxs[b], isems[b])

    def _drain_idx(b):
        pltpu.make_async_copy(srcs_hbm.at[pl.ds(0, K)], sidxs[b],
                              isems[b]).wait()
        pltpu.make_async_copy(dsts_hbm.at[pl.ds(0, K)], didxs[b],
                              isems[b]).wait()

    def _issue_gather(b):
        pltpu.async_copy(stab_hbm.at[sidxs[b]], sgs[b], gsems[b])
        pltpu.async_copy(dtab_hbm.at[didxs[b]], dgs[b], gsems[b])

    def _drain_gather(b):
        pltpu.make_async_copy(stab_hbm.at[pl.ds(0, K)], sgs[b],
                              gsems[b]).wait()
        pltpu.make_async_copy(dtab_hbm.at[pl.ds(0, K)], dgs[b],
                              gsems[b]).wait()

    ROT = lax.iota(jnp.int32, 16) ^ 8

    _issue_idx(0, 0)
    _drain_idx(0)
    _issue_gather(0)
    _issue_idx(1, 1)

    def _outer(gg, _):
        for b in range(2):
            g = 2 * gg + b
            sg = sgs[b]
            dg = dgs[b]

            @pl.when(g + 1 < NCH)
            def _pf():
                _drain_idx(1 - b)
                _issue_gather(1 - b)

            _drain_gather(b)

            @plsc.parallel_loop(0, K, 1, unroll=4)
            def _edge(i):
                asd = sg[i, pl.ds(HF, 16)]
                drow = dg[i, :]
                tt = asd + drow
                e = jnp.where(tt > 0, tt, NEG * tt)
                rot = _lane_gather(drow, ROT)
                ex = jnp.exp(e - rot)
                sg[i, pl.ds(HF, 16)] = ex
                for j in range(H):
                    sl = pl.ds(16 * j, 16)
                    sg[i, sl] = sg[i, sl] * _bcast_lane(ex, j)

            pltpu.sync_copy(sg, out_acc.at[didxs[b]], add=True)

            @pl.when(g + 2 < NCH)
            def _pfidx():
                _issue_idx(g + 2, b)
        return _
    lax.fori_loop(0, NCH // 2, _outer, None)

    plsc.subcore_barrier()
    pltpu.sync_copy(out_acc.at[pl.ds(r0, ROWS_PER_TILE)],
                    outp_hbm.at[c, pl.ds(r0, ROWS_PER_TILE)])


def _sc_edge_pass(stab, dtab, srcs, dsts):
    mesh = plsc.VectorSubcoreMesh(core_axis_name="c", subcore_axis_name="s")
    run = functools.partial(
        pl.kernel,
        mesh=mesh,
        compiler_params=pltpu.CompilerParams(use_tc_tiling_on_sc=False),
        out_type=jax.ShapeDtypeStruct((2, NP, TW), jnp.float32),
        scratch_types=[
            pltpu.VMEM((K,), jnp.int32),
            pltpu.VMEM((K,), jnp.int32),
            pltpu.VMEM((K,), jnp.int32),
            pltpu.VMEM((K,), jnp.int32),
            pltpu.VMEM((K, TW), jnp.float32),
            pltpu.VMEM((K, TW), jnp.float32),
            pltpu.VMEM((K, 16), jnp.float32),
            pltpu.VMEM((K, 16), jnp.float32),
            pltpu.SemaphoreType.DMA,
            pltpu.SemaphoreType.DMA,
            pltpu.SemaphoreType.DMA,
            pltpu.SemaphoreType.DMA,
            pltpu.VMEM_SHARED((NP, TW), jnp.float32),
        ],
    )(_sc_edge_kernel)
    return run(stab, dtab, srcs, dsts)


def _post_body(outp_ref, h_ref, bias_ref, gamma_ref, beta_ref,
               b128_ref, o_ref):
    acc = (outp_ref[0, :N, :HF] + outp_ref[1, :N, :HF] + h_ref[...])
    den = (outp_ref[0, :N, HF:HF + 16] + outp_ref[1, :N, HF:HF + 16]
           + (1.0 + 1e-16))
    dinv = 1.0 / den
    dinv128 = jnp.dot(dinv, b128_ref[...],
                      preferred_element_type=jnp.float32)
    y = acc * dinv128 + bias_ref[...]
    mean = jnp.mean(y, axis=0, keepdims=True)
    var = jnp.mean((y - mean) ** 2, axis=0, keepdims=True)
    yn = (y - mean) / jnp.sqrt(var + 1e-5) * gamma_ref[...] + beta_ref[...]
    o_ref[...] = jnp.where(yn > 0, yn, NEG * yn)


def _tc_epilogue(outp, h, bias, gamma, beta, B128):
    return pl.pallas_call(
        _post_body,
        out_shape=jax.ShapeDtypeStruct((N, HF), jnp.float32),
    )(outp, h, bias, gamma, beta, B128)


def kernel(x, edge_index, W, a_src, a_dst, bias, gamma, beta):
    r = jnp.arange(HF, dtype=jnp.int32)
    Ms = jnp.zeros((HF, H), jnp.float32).at[r, r // F].set(a_src.reshape(-1))
    Md = jnp.zeros((HF, H), jnp.float32).at[r, r // F].set(a_dst.reshape(-1))
    B128 = jnp.zeros((16, HF), jnp.float32).at[r // F, r].set(1.0)

    stab, dtab = _tc_prologue(x, W, Ms, Md)
    h = lax.slice(stab, (0, 0), (N, HF))

    pad = EP - E
    srcs = jnp.concatenate([edge_index[0], jnp.zeros((pad,), jnp.int32)])
    dsts = jnp.concatenate([edge_index[1], jnp.full((pad,), N, jnp.int32)])

    outp = _sc_edge_pass(stab, dtab, srcs, dsts)

    return _tc_epilogue(outp, h, bias.reshape(1, HF),
                        gamma.reshape(1, HF), beta.reshape(1, HF), B128)
